# baseline (speedup 1.0000x reference)
import math

import jax
import jax.numpy as jnp
from jax import lax
from jax.experimental import pallas as pl
from jax.experimental.pallas import tpu as pltpu
from jax.experimental.pallas import tpu_sc as plsc

VOCAB = 100000
EMB = 128
SCALE = math.sqrt(float(EMB))

NC = 2
NS = 16
NW = NC * NS

NTOK = 4096
SEQ = 50
TBLK = NTOK // NW
NBUF = 5
NGRP = SEQ // NBUF


def _sc_body(tokT_hbm, table_hbm, out_hbm, idx_v, bufs, *sems):
    gsem = sems[:NBUF]
    ssem = sems[NBUF:]
    wid = lax.axis_index("s") * NC + lax.axis_index("c")
    t0 = pl.multiple_of(wid * TBLK, TBLK)
    pltpu.sync_copy(tokT_hbm.at[:, pl.ds(t0, TBLK)], idx_v)

    def gather_start(s, b):
        pltpu.async_copy(table_hbm.at[idx_v.at[s]], bufs.at[b], gsem[b])

    def gather_wait(s, b):
        pltpu.make_async_copy(table_hbm.at[idx_v.at[s]], bufs.at[b], gsem[b]).wait()

    def out_slot(s):
        return out_hbm.at[s, pl.ds(t0, TBLK)]

    def scatter_start(s, b):
        pltpu.async_copy(bufs.at[b], out_slot(s), ssem[b])

    def scatter_wait(s, b):
        pltpu.make_async_copy(bufs.at[b], out_slot(s), ssem[b]).wait()

    def scale_buf(b):
        def row_body(r, carry):
            for rr in range(4):
                for j in range(EMB // 16):
                    sl = pl.ds(j * 16, 16)
                    bufs[b, r * 4 + rr, sl] = bufs[b, r * 4 + rr, sl] * SCALE
            return carry

        lax.fori_loop(0, TBLK // 4, row_body, 0)

    for b in range(NBUF):
        gather_start(b, b)

    def group_body(g, carry):
        sg = g * NBUF
        for b in range(NBUF):
            gather_wait(sg + b, b)
            scale_buf(b)
            scatter_start(sg + b, b)
        for b in range(NBUF):
            scatter_wait(sg + b, b)
            gather_start(sg + NBUF + b, b)
        return carry

    lax.fori_loop(0, NGRP - 1, group_body, 0)

    sg = (NGRP - 1) * NBUF
    for b in range(NBUF):
        gather_wait(sg + b, b)
        scale_buf(b)
        scatter_start(sg + b, b)
    for b in range(NBUF):
        scatter_wait(sg + b, b)


@jax.jit
def _sc_embed(tokens_t, table):
    mesh = plsc.VectorSubcoreMesh(core_axis_name="c", subcore_axis_name="s")
    run = pl.kernel(
        _sc_body,
        out_type=jax.ShapeDtypeStruct((SEQ, NTOK, EMB), jnp.float32),
        mesh=mesh,
        scratch_types=[
            pltpu.VMEM((SEQ, TBLK), jnp.int32),
            pltpu.VMEM((NBUF, TBLK, EMB), jnp.float32),
        ] + [pltpu.SemaphoreType.DMA] * (2 * NBUF),
    )
    return run(tokens_t, table)


def kernel(tokens, table):
    out5 = _sc_embed(tokens.T, table)
    return jnp.transpose(out5, (1, 0, 2))


# speedup vs baseline: 1.0011x; 1.0011x over previous
import math

import jax
import jax.numpy as jnp
from jax import lax
from jax.experimental import pallas as pl
from jax.experimental.pallas import tpu as pltpu
from jax.experimental.pallas import tpu_sc as plsc

VOCAB = 100000
EMB = 128
SCALE = math.sqrt(float(EMB))

NC = 2
NS = 16
NW = NC * NS

NTOK = 4096
SEQ = 50
TBLK = NTOK // NW
NBUF = 5
NGRP = SEQ // NBUF


def _sc_body(tokT_hbm, table_hbm, out_hbm, idx_v, bufs, *sems):
    gsem = sems[:NBUF]
    ssem = sems[NBUF:]
    wid = lax.axis_index("s") * NC + lax.axis_index("c")
    t0 = pl.multiple_of(wid * TBLK, TBLK)
    pltpu.sync_copy(tokT_hbm.at[:, pl.ds(t0, TBLK)], idx_v)

    def gather_start(s, b):
        pltpu.async_copy(table_hbm.at[idx_v.at[s]], bufs.at[b], gsem[b])

    def gather_wait(s, b):
        pltpu.make_async_copy(table_hbm.at[idx_v.at[s]], bufs.at[b], gsem[b]).wait()

    def out_slot(s):
        return out_hbm.at[s, pl.ds(t0, TBLK)]

    def scatter_start(s, b):
        pltpu.async_copy(bufs.at[b], out_slot(s), ssem[b])

    def scatter_wait(s, b):
        pltpu.make_async_copy(bufs.at[b], out_slot(s), ssem[b]).wait()

    def scale_buf(b):
        def row_body(r, carry):
            for rr in range(4):
                for j in range(EMB // 16):
                    sl = pl.ds(j * 16, 16)
                    bufs[b, r * 4 + rr, sl] = bufs[b, r * 4 + rr, sl] * SCALE
            return carry

        lax.fori_loop(0, TBLK // 4, row_body, 0)

    for b in range(NBUF):
        gather_start(b, b)

    def group_body(g, carry):
        sg = g * NBUF
        for b in range(NBUF):
            gather_wait(sg + b, b)
            scale_buf(b)
            scatter_start(sg + b, b)
        for b in range(NBUF):
            scatter_wait(sg + b, b)
            gather_start(sg + NBUF + b, b)
        return carry

    lax.fori_loop(0, NGRP - 1, group_body, 0)

    sg = (NGRP - 1) * NBUF
    for b in range(NBUF):
        gather_wait(sg + b, b)
        scale_buf(b)
        scatter_start(sg + b, b)
    for b in range(NBUF):
        scatter_wait(sg + b, b)


@jax.jit
def _sc_embed(tokens_t, table):
    mesh = plsc.VectorSubcoreMesh(core_axis_name="c", subcore_axis_name="s")
    run = pl.kernel(
        _sc_body,
        out_type=jax.ShapeDtypeStruct((SEQ, NTOK, EMB), jnp.float32),
        mesh=mesh,
        scratch_types=[
            pltpu.VMEM((SEQ, TBLK), jnp.int32),
            pltpu.VMEM((NBUF, TBLK, EMB), jnp.float32),
        ] + [pltpu.SemaphoreType.DMA] * (2 * NBUF),
    )
    return run(tokens_t, table)


def kernel(tokens, table):
    out5 = _sc_embed(tokens.astype(jnp.int32).T, table)
    return jnp.transpose(out5, (1, 0, 2))
